# baseline (speedup 1.0000x reference)
import functools

import jax
import jax.numpy as jnp
from jax import lax
from jax.experimental import pallas as pl
from jax.experimental.pallas import tpu as pltpu
from jax.experimental.pallas import tpu_sc as plsc

DA = 144
CH = 128
NROWS_PAD = 10240


def _sc_aggregate(xa, colp, rowp, kpw0, kpw1):
    info = plsc.get_sparse_core_info()
    nc, ns = info.num_cores, info.num_subcores
    rows_per_tile = NROWS_PAD // ns
    mesh = plsc.VectorSubcoreMesh(core_axis_name="c", subcore_axis_name="s")

    @functools.partial(
        pl.kernel,
        mesh=mesh,
        compiler_params=pltpu.CompilerParams(use_tc_tiling_on_sc=False),
        out_type=jax.ShapeDtypeStruct((nc, NROWS_PAD, DA), jnp.float32),
        scratch_types=(
            [pltpu.VMEM((CH,), jnp.int32)] * 4
            + [pltpu.VMEM((CH,), jnp.int32)] * 4
            + [pltpu.VMEM((CH, DA), jnp.float32)] * 2
            + [pltpu.VMEM_SHARED((NROWS_PAD, DA), jnp.float32)]
            + [pltpu.SemaphoreType.DMA] * 8
        ),
    )
    def k(xa_hbm, col_hbm, row_hbm, out_hbm,
          cc0, cc1, cc2, cc3, rc0, rc1, rc2, rc3, rows0, rows1, agg,
          is0, is1, is2, is3, gsem0, gsem1, ssem0, ssem1):
        c = lax.axis_index("c")
        s = lax.axis_index("s")
        kpw = jnp.where(c == 0, kpw0, kpw1)
        base_chunk = c * ns * kpw0 + s * kpw
        colc = (cc0, cc1, cc2, cc3)
        rowc = (rc0, rc1, rc2, rc3)
        rows = (rows0, rows1)
        isem = (is0, is1, is2, is3)
        gsem = (gsem0, gsem1)
        ssem = (ssem0, ssem1)

        def zrow(i, carry):
            for j in range(DA // 16):
                rows0[i, pl.ds(j * 16, 16)] = jnp.zeros((16,), jnp.float32)
            return carry

        lax.fori_loop(0, CH, zrow, 0)
        for t in range(rows_per_tile // CH):
            pltpu.sync_copy(rows0, agg.at[pl.ds(s * rows_per_tile + t * CH, CH)])
        plsc.subcore_barrier()

        base_e = base_chunk * CH

        def idx_fetch(g, i):
            e0 = base_e + g * CH
            pltpu.async_copy(col_hbm.at[pl.ds(e0, CH)], colc[i], isem[i])
            pltpu.async_copy(row_hbm.at[pl.ds(e0, CH)], rowc[i], isem[i])

        def idx_wait(i):
            pltpu.make_async_copy(
                col_hbm.at[pl.ds(base_e, CH)], colc[i], isem[i]).wait()
            pltpu.make_async_copy(
                row_hbm.at[pl.ds(base_e, CH)], rowc[i], isem[i]).wait()

        def gather_launch(p, i):
            pltpu.async_copy(xa_hbm.at[colc[i]], rows[p], gsem[p])

        def gather_wait(p, i):
            pltpu.make_async_copy(
                xa_hbm.at[colc[i]], rows[p], gsem[p]).wait()

        def scatter_launch(p, i):
            pltpu.async_copy(rows[p], agg.at[rowc[i]], ssem[p], add=True)

        def scatter_wait(p, i):
            pltpu.make_async_copy(
                rows[p], agg.at[rowc[i]], ssem[p]).wait()

        idx_fetch(0, 0)
        idx_fetch(1, 1)
        idx_wait(0)
        gather_launch(0, 0)

        def body(h, carry):
            for q in range(4):
                g = 4 * h + q
                p = q % 2
                i = q

                @pl.when((g + 1 < kpw) & (g >= 1))
                def _():
                    scatter_wait(1 - p, (i + 3) % 4)

                @pl.when(g + 1 < kpw)
                def _():
                    idx_wait((i + 1) % 4)
                    gather_launch(1 - p, (i + 1) % 4)

                gather_wait(p, i)
                scatter_launch(p, i)

                @pl.when(g + 2 < kpw)
                def _():
                    idx_fetch(g + 2, (i + 2) % 4)
            return carry

        lax.fori_loop(0, kpw // 4, body, 0)
        scatter_wait(0, 0)
        scatter_wait(1, 0)
        plsc.subcore_barrier()
        pltpu.sync_copy(
            agg.at[pl.ds(s * rows_per_tile, rows_per_tile)],
            out_hbm.at[c, pl.ds(s * rows_per_tile, rows_per_tile)],
        )

    return k(xa, colp, rowp)


def _tc_finish_body(xa_ref, p_ref, wa_ref, g_ref, b_ref, o_ref):
    s = xa_ref[...] + p_ref[0] + p_ref[1]
    h = lax.dot_general(
        s, wa_ref[...], (((1,), (1,)), ((), ())),
        preferred_element_type=jnp.float32,
    )
    mean = jnp.mean(h, axis=1, keepdims=True)
    d = h - mean
    var = jnp.mean(d * d, axis=1, keepdims=True)
    y = d * lax.rsqrt(var + 1e-5) * g_ref[...] + b_ref[...]
    o_ref[...] = jnp.maximum(y, 0.0)


def _tc_finish(xa, parts, wa, gamma2, beta2):
    n = xa.shape[0]
    dout = wa.shape[0]
    bs = 2000
    grid = n // bs
    return pl.pallas_call(
        _tc_finish_body,
        grid=(grid,),
        in_specs=[
            pl.BlockSpec((bs, DA), lambda i: (i, 0)),
            pl.BlockSpec((2, bs, DA), lambda i: (0, i, 0)),
            pl.BlockSpec((dout, DA), lambda i: (0, 0)),
            pl.BlockSpec((1, dout), lambda i: (0, 0)),
            pl.BlockSpec((1, dout), lambda i: (0, 0)),
        ],
        out_specs=pl.BlockSpec((bs, dout), lambda i: (i, 0)),
        out_shape=jax.ShapeDtypeStruct((n, dout), jnp.float32),
    )(xa, parts, wa, gamma2, beta2)


def kernel(x, edge_index, W, b, gamma, beta):
    n, d_in = x.shape
    d_out = W.shape[0]
    e = edge_index.shape[1]
    row = edge_index[0].astype(jnp.int32)
    col = edge_index[1].astype(jnp.int32)

    xa = jnp.concatenate(
        [x, jnp.ones((n, 1), jnp.float32), jnp.zeros((n, DA - d_in - 1), jnp.float32)],
        axis=1,
    )
    wa = jnp.concatenate(
        [W, b[:, None], jnp.zeros((d_out, DA - d_in - 1), jnp.float32)], axis=1
    )

    info = plsc.get_sparse_core_info()
    ns = info.num_subcores
    tot = -(-e // CH)
    kpw0 = kpw1 = (-(-tot // (2 * ns)) + 3) // 4 * 4
    e_pad = ns * (kpw0 + kpw1) * CH
    npad = e_pad - e
    colp = jnp.concatenate([col, jnp.zeros((npad,), jnp.int32)])
    rowp = jnp.concatenate(
        [row, n + jnp.arange(npad, dtype=jnp.int32) % (NROWS_PAD - n)])

    parts = _sc_aggregate(xa, colp, rowp, kpw0, kpw1)
    return _tc_finish(xa, parts, wa, gamma.reshape(1, d_out), beta.reshape(1, d_out))


# speedup vs baseline: 2.8764x; 2.8764x over previous
import functools

import jax
import jax.numpy as jnp
from jax import lax
from jax.experimental import pallas as pl
from jax.experimental.pallas import tpu as pltpu
from jax.experimental.pallas import tpu_sc as plsc

DA = 144
CH = 128
NROWS_PAD = 10240


def _sc_aggregate(xa, colp, rowp, kpw):
    info = plsc.get_sparse_core_info()
    nc, ns = info.num_cores, info.num_subcores
    rows_per_tile = NROWS_PAD // ns
    mesh = plsc.VectorSubcoreMesh(core_axis_name="c", subcore_axis_name="s")

    @functools.partial(
        pl.kernel,
        mesh=mesh,
        compiler_params=pltpu.CompilerParams(use_tc_tiling_on_sc=False),
        out_type=jax.ShapeDtypeStruct((nc, NROWS_PAD, DA), jnp.float32),
        scratch_types=(
            [pltpu.VMEM((CH,), jnp.int32)] * 4
            + [pltpu.VMEM((CH,), jnp.int32)] * 4
            + [pltpu.VMEM((CH, DA), jnp.float32)] * 2
            + [pltpu.VMEM_SHARED((NROWS_PAD, DA), jnp.float32)]
            + [pltpu.SemaphoreType.DMA] * 8
        ),
    )
    def k(xa_hbm, col_hbm, row_hbm, out_hbm,
          cc0, cc1, cc2, cc3, rc0, rc1, rc2, rc3, rows0, rows1, agg,
          is0, is1, is2, is3, gsem0, gsem1, ssem0, ssem1):
        c = lax.axis_index("c")
        s = lax.axis_index("s")
        wid = c * ns + s
        nw = nc * ns
        colc = (cc0, cc1, cc2, cc3)
        rowc = (rc0, rc1, rc2, rc3)
        rows = (rows0, rows1)
        isem = (is0, is1, is2, is3)
        gsem = (gsem0, gsem1)
        ssem = (ssem0, ssem1)

        def zrow(i, carry):
            for j in range(DA // 16):
                rows0[i, pl.ds(j * 16, 16)] = jnp.zeros((16,), jnp.float32)
            return carry

        lax.fori_loop(0, CH, zrow, 0)
        for t in range(rows_per_tile // CH):
            pltpu.sync_copy(rows0, agg.at[pl.ds(s * rows_per_tile + t * CH, CH)])
        plsc.subcore_barrier()

        def idx_fetch(g, i):
            e0 = (wid + g * nw) * CH
            pltpu.async_copy(col_hbm.at[pl.ds(e0, CH)], colc[i], isem[i])
            pltpu.async_copy(row_hbm.at[pl.ds(e0, CH)], rowc[i], isem[i])

        def idx_wait(i):
            pltpu.make_async_copy(
                col_hbm.at[pl.ds(0, CH)], colc[i], isem[i]).wait()
            pltpu.make_async_copy(
                row_hbm.at[pl.ds(0, CH)], rowc[i], isem[i]).wait()

        def gather_launch(p, i):
            pltpu.async_copy(xa_hbm.at[colc[i]], rows[p], gsem[p])

        def gather_wait(p, i):
            pltpu.make_async_copy(
                xa_hbm.at[colc[i]], rows[p], gsem[p]).wait()

        def scatter_launch(p, i):
            pltpu.async_copy(rows[p], agg.at[rowc[i]], ssem[p], add=True)

        def scatter_wait(p, i):
            pltpu.make_async_copy(
                rows[p], agg.at[rowc[i]], ssem[p]).wait()

        idx_fetch(0, 0)
        idx_fetch(1, 1)
        idx_wait(0)
        gather_launch(0, 0)

        def body(h, carry):
            for q in range(4):
                g = 4 * h + q
                p = q % 2
                i = q

                @pl.when((g + 1 < kpw) & (g >= 1))
                def _():
                    scatter_wait(1 - p, (i + 3) % 4)

                @pl.when(g + 1 < kpw)
                def _():
                    idx_wait((i + 1) % 4)
                    gather_launch(1 - p, (i + 1) % 4)

                gather_wait(p, i)
                scatter_launch(p, i)

                @pl.when(g + 2 < kpw)
                def _():
                    idx_fetch(g + 2, (i + 2) % 4)
            return carry

        lax.fori_loop(0, kpw // 4, body, 0)
        scatter_wait(0, 0)
        scatter_wait(1, 0)
        plsc.subcore_barrier()
        pltpu.sync_copy(
            agg.at[pl.ds(s * rows_per_tile, rows_per_tile)],
            out_hbm.at[c, pl.ds(s * rows_per_tile, rows_per_tile)],
        )

    return k(xa, colp, rowp)


def _tc_finish_body(xa_ref, p_ref, wa_ref, g_ref, b_ref, o_ref):
    s = xa_ref[...] + p_ref[0] + p_ref[1]
    h = lax.dot_general(
        s, wa_ref[...], (((1,), (1,)), ((), ())),
        preferred_element_type=jnp.float32,
    )
    mean = jnp.mean(h, axis=1, keepdims=True)
    d = h - mean
    var = jnp.mean(d * d, axis=1, keepdims=True)
    y = d * lax.rsqrt(var + 1e-5) * g_ref[...] + b_ref[...]
    o_ref[...] = jnp.maximum(y, 0.0)


def _tc_finish(xa, parts, wa, gamma2, beta2):
    n = xa.shape[0]
    dout = wa.shape[0]
    bs = 2000
    grid = n // bs
    return pl.pallas_call(
        _tc_finish_body,
        grid=(grid,),
        in_specs=[
            pl.BlockSpec((bs, DA), lambda i: (i, 0)),
            pl.BlockSpec((2, bs, DA), lambda i: (0, i, 0)),
            pl.BlockSpec((dout, DA), lambda i: (0, 0)),
            pl.BlockSpec((1, dout), lambda i: (0, 0)),
            pl.BlockSpec((1, dout), lambda i: (0, 0)),
        ],
        out_specs=pl.BlockSpec((bs, dout), lambda i: (i, 0)),
        out_shape=jax.ShapeDtypeStruct((n, dout), jnp.float32),
    )(xa, parts, wa, gamma2, beta2)


def kernel(x, edge_index, W, b, gamma, beta):
    n, d_in = x.shape
    d_out = W.shape[0]
    e = edge_index.shape[1]
    row = edge_index[0].astype(jnp.int32)
    col = edge_index[1].astype(jnp.int32)

    xa = jnp.concatenate(
        [x, jnp.ones((n, 1), jnp.float32), jnp.zeros((n, DA - d_in - 1), jnp.float32)],
        axis=1,
    )
    wa = jnp.concatenate(
        [W, b[:, None], jnp.zeros((d_out, DA - d_in - 1), jnp.float32)], axis=1
    )

    info = plsc.get_sparse_core_info()
    ns = info.num_subcores
    tot = -(-e // CH)
    kpw = (-(-tot // (2 * ns)) + 3) // 4 * 4
    e_pad = 2 * ns * kpw * CH
    npad = e_pad - e
    ar = jnp.arange(npad, dtype=jnp.int32)
    colp = jnp.concatenate([col, (ar * 131) % n])
    rowp = jnp.concatenate([row, n + ar % (NROWS_PAD - n)])

    parts = _sc_aggregate(xa, colp, rowp, kpw)
    return _tc_finish(xa, parts, wa, gamma.reshape(1, d_out), beta.reshape(1, d_out))


# speedup vs baseline: 3.0420x; 1.0576x over previous
import functools

import jax
import jax.numpy as jnp
from jax import lax
from jax.experimental import pallas as pl
from jax.experimental.pallas import tpu as pltpu
from jax.experimental.pallas import tpu_sc as plsc

DA = 144
CH = 128
NROWS_PAD = 10240


def _sc_aggregate(xa, ei, pad_ei, kpw, tot_real):
    info = plsc.get_sparse_core_info()
    nc, ns = info.num_cores, info.num_subcores
    rows_per_tile = NROWS_PAD // ns
    mesh = plsc.VectorSubcoreMesh(core_axis_name="c", subcore_axis_name="s")

    @functools.partial(
        pl.kernel,
        mesh=mesh,
        compiler_params=pltpu.CompilerParams(use_tc_tiling_on_sc=False),
        out_type=jax.ShapeDtypeStruct((nc, NROWS_PAD, DA), jnp.float32),
        scratch_types=(
            [pltpu.VMEM((CH,), jnp.int32)] * 4
            + [pltpu.VMEM((CH,), jnp.int32)] * 4
            + [pltpu.VMEM((CH, DA), jnp.float32)] * 2
            + [pltpu.VMEM_SHARED((NROWS_PAD, DA), jnp.float32)]
            + [pltpu.SemaphoreType.DMA] * 8
        ),
    )
    def k(xa_hbm, ei_hbm, pad_hbm, out_hbm,
          cc0, cc1, cc2, cc3, rc0, rc1, rc2, rc3, rows0, rows1, agg,
          is0, is1, is2, is3, gsem0, gsem1, ssem0, ssem1):
        c = lax.axis_index("c")
        s = lax.axis_index("s")
        wid = c * ns + s
        nw = nc * ns
        colc = (cc0, cc1, cc2, cc3)
        rowc = (rc0, rc1, rc2, rc3)
        rows = (rows0, rows1)
        isem = (is0, is1, is2, is3)
        gsem = (gsem0, gsem1)
        ssem = (ssem0, ssem1)

        def zrow(i, carry):
            for j in range(DA // 16):
                rows0[i, pl.ds(j * 16, 16)] = jnp.zeros((16,), jnp.float32)
            return carry

        lax.fori_loop(0, CH, zrow, 0)
        for t in range(rows_per_tile // CH):
            pltpu.sync_copy(rows0, agg.at[pl.ds(s * rows_per_tile + t * CH, CH)])
        plsc.subcore_barrier()

        def idx_fetch(g, i):
            cid = wid + g * nw

            @pl.when(cid < tot_real)
            def _():
                e0 = cid * CH
                pltpu.async_copy(ei_hbm.at[1, pl.ds(e0, CH)], colc[i], isem[i])
                pltpu.async_copy(ei_hbm.at[0, pl.ds(e0, CH)], rowc[i], isem[i])

            @pl.when(cid >= tot_real)
            def _():
                e0 = (cid - tot_real) * CH
                pltpu.async_copy(pad_hbm.at[1, pl.ds(e0, CH)], colc[i], isem[i])
                pltpu.async_copy(pad_hbm.at[0, pl.ds(e0, CH)], rowc[i], isem[i])

        def idx_wait(i):
            pltpu.make_async_copy(
                ei_hbm.at[1, pl.ds(0, CH)], colc[i], isem[i]).wait()
            pltpu.make_async_copy(
                ei_hbm.at[0, pl.ds(0, CH)], rowc[i], isem[i]).wait()

        def gather_launch(p, i):
            pltpu.async_copy(xa_hbm.at[colc[i]], rows[p], gsem[p])

        def gather_wait(p, i):
            pltpu.make_async_copy(
                xa_hbm.at[colc[i]], rows[p], gsem[p]).wait()

        def scatter_launch(p, i):
            pltpu.async_copy(rows[p], agg.at[rowc[i]], ssem[p], add=True)

        def scatter_wait(p, i):
            pltpu.make_async_copy(
                rows[p], agg.at[rowc[i]], ssem[p]).wait()

        idx_fetch(0, 0)
        idx_fetch(1, 1)
        idx_wait(0)
        gather_launch(0, 0)

        def body(h, carry):
            for q in range(4):
                g = 4 * h + q
                p = q % 2
                i = q

                @pl.when((g + 1 < kpw) & (g >= 1))
                def _():
                    scatter_wait(1 - p, (i + 3) % 4)

                @pl.when(g + 1 < kpw)
                def _():
                    idx_wait((i + 1) % 4)
                    gather_launch(1 - p, (i + 1) % 4)

                gather_wait(p, i)
                scatter_launch(p, i)

                @pl.when(g + 2 < kpw)
                def _():
                    idx_fetch(g + 2, (i + 2) % 4)
            return carry

        lax.fori_loop(0, kpw // 4, body, 0)
        scatter_wait(0, 0)
        scatter_wait(1, 0)
        plsc.subcore_barrier()
        pltpu.sync_copy(
            agg.at[pl.ds(s * rows_per_tile, rows_per_tile)],
            out_hbm.at[c, pl.ds(s * rows_per_tile, rows_per_tile)],
        )

    return k(xa, ei, pad_ei)


def _tc_finish_body(xa_ref, p_ref, wa_ref, g_ref, b_ref, o_ref):
    s = xa_ref[...] + p_ref[0] + p_ref[1]
    h = lax.dot_general(
        s, wa_ref[...], (((1,), (1,)), ((), ())),
        preferred_element_type=jnp.float32,
    )
    mean = jnp.mean(h, axis=1, keepdims=True)
    d = h - mean
    var = jnp.mean(d * d, axis=1, keepdims=True)
    y = d * lax.rsqrt(var + 1e-5) * g_ref[...] + b_ref[...]
    o_ref[...] = jnp.maximum(y, 0.0)


def _tc_finish(xa, parts, wa, gamma2, beta2):
    n = xa.shape[0]
    dout = wa.shape[0]
    bs = 2000
    grid = n // bs
    return pl.pallas_call(
        _tc_finish_body,
        grid=(grid,),
        in_specs=[
            pl.BlockSpec((bs, DA), lambda i: (i, 0)),
            pl.BlockSpec((2, bs, DA), lambda i: (0, i, 0)),
            pl.BlockSpec((dout, DA), lambda i: (0, 0)),
            pl.BlockSpec((1, dout), lambda i: (0, 0)),
            pl.BlockSpec((1, dout), lambda i: (0, 0)),
        ],
        out_specs=pl.BlockSpec((bs, dout), lambda i: (i, 0)),
        out_shape=jax.ShapeDtypeStruct((n, dout), jnp.float32),
    )(xa, parts, wa, gamma2, beta2)


def kernel(x, edge_index, W, b, gamma, beta):
    n, d_in = x.shape
    d_out = W.shape[0]
    e = edge_index.shape[1]
    ei = edge_index.astype(jnp.int32)

    xa = jnp.concatenate(
        [x, jnp.ones((n, 1), jnp.float32), jnp.zeros((n, DA - d_in - 1), jnp.float32)],
        axis=1,
    )
    wa = jnp.concatenate(
        [W, b[:, None], jnp.zeros((d_out, DA - d_in - 1), jnp.float32)], axis=1
    )

    info = plsc.get_sparse_core_info()
    ns = info.num_subcores
    tot = -(-e // CH)
    kpw = (-(-tot // (2 * ns)) + 3) // 4 * 4
    e_pad = 2 * ns * kpw * CH
    tot_real = e // CH
    npe = e_pad - tot_real * CH
    nsynth = e_pad - e
    ar = jnp.arange(nsynth, dtype=jnp.int32)
    pad_col = jnp.concatenate([ei[1, tot_real * CH:], (ar * 131) % n])
    pad_row = jnp.concatenate(
        [ei[0, tot_real * CH:], n + ar % (NROWS_PAD - n)])
    pad_ei = jnp.stack([pad_row, pad_col])
    assert pad_ei.shape == (2, npe)

    parts = _sc_aggregate(xa, ei, pad_ei, kpw, tot_real)
    return _tc_finish(xa, parts, wa, gamma.reshape(1, d_out), beta.reshape(1, d_out))


# speedup vs baseline: 3.2345x; 1.0633x over previous
import functools

import jax
import jax.numpy as jnp
from jax import lax
from jax.experimental import pallas as pl
from jax.experimental.pallas import tpu as pltpu
from jax.experimental.pallas import tpu_sc as plsc

D = 128
CH = 128
DW = 16
NROWS_PAD = 10240


def _sc_aggregate(x, row1, col1, prow1, pcol1, z128, z16, ones, kpw, tot_real):
    info = plsc.get_sparse_core_info()
    nc, ns = info.num_cores, info.num_subcores
    rows_per_tile = NROWS_PAD // ns
    mesh = plsc.VectorSubcoreMesh(core_axis_name="c", subcore_axis_name="s")

    @functools.partial(
        pl.kernel,
        mesh=mesh,
        compiler_params=pltpu.CompilerParams(use_tc_tiling_on_sc=False),
        out_type=[
            jax.ShapeDtypeStruct((nc, NROWS_PAD, D), jnp.float32),
            jax.ShapeDtypeStruct((nc, NROWS_PAD, DW), jnp.float32),
        ],
        scratch_types=(
            [pltpu.VMEM((CH,), jnp.int32)] * 4
            + [pltpu.VMEM((CH,), jnp.int32)] * 4
            + [pltpu.VMEM((CH, D), jnp.float32)] * 2
            + [pltpu.VMEM((CH, DW), jnp.float32)]
            + [pltpu.VMEM_SHARED((NROWS_PAD, D), jnp.float32)]
            + [pltpu.VMEM_SHARED((NROWS_PAD, DW), jnp.float32)]
            + [pltpu.SemaphoreType.DMA] * 10
        ),
    )
    def k(x_hbm, row_hbm, col_hbm, prow_hbm, pcol_hbm, z128_hbm, z16_hbm,
          ones_hbm, agg_hbm, deg_hbm,
          cc0, cc1, cc2, cc3, rc0, rc1, rc2, rc3, rows0, rows1, ones_v,
          agg, degsp,
          is0, is1, is2, is3, gsem0, gsem1, ssem0, ssem1, dsem0, dsem1):
        c = lax.axis_index("c")
        s = lax.axis_index("s")
        wid = c * ns + s
        nw = nc * ns
        colc = (cc0, cc1, cc2, cc3)
        rowc = (rc0, rc1, rc2, rc3)
        rows = (rows0, rows1)
        isem = (is0, is1, is2, is3)
        gsem = (gsem0, gsem1)
        ssem = (ssem0, ssem1)
        dsem = (dsem0, dsem1)

        for t in range(rows_per_tile // CH):
            pltpu.sync_copy(
                z128_hbm, agg.at[pl.ds(s * rows_per_tile + t * CH, CH)])
        pltpu.sync_copy(z16_hbm, degsp.at[pl.ds(s * rows_per_tile, rows_per_tile)])
        pltpu.sync_copy(ones_hbm, ones_v)
        plsc.subcore_barrier()

        def idx_fetch(g, i):
            cid = wid + g * nw

            @pl.when(cid < tot_real)
            def _():
                e0 = cid * CH
                pltpu.async_copy(col_hbm.at[pl.ds(e0, CH)], colc[i], isem[i])
                pltpu.async_copy(row_hbm.at[pl.ds(e0, CH)], rowc[i], isem[i])

            @pl.when(cid >= tot_real)
            def _():
                e0 = (cid - tot_real) * CH
                pltpu.async_copy(pcol_hbm.at[pl.ds(e0, CH)], colc[i], isem[i])
                pltpu.async_copy(prow_hbm.at[pl.ds(e0, CH)], rowc[i], isem[i])

        def idx_wait(i):
            pltpu.make_async_copy(
                col_hbm.at[pl.ds(0, CH)], colc[i], isem[i]).wait()
            pltpu.make_async_copy(
                row_hbm.at[pl.ds(0, CH)], rowc[i], isem[i]).wait()

        def gather_launch(p, i):
            pltpu.async_copy(x_hbm.at[colc[i]], rows[p], gsem[p])

        def gather_wait(p, i):
            pltpu.make_async_copy(
                x_hbm.at[colc[i]], rows[p], gsem[p]).wait()

        def scatter_launch(p, i):
            pltpu.async_copy(rows[p], agg.at[rowc[i]], ssem[p], add=True)
            pltpu.async_copy(ones_v, degsp.at[rowc[i]], dsem[p], add=True)

        def scatter_wait(p, i):
            pltpu.make_async_copy(
                rows[p], agg.at[rowc[i]], ssem[p]).wait()
            pltpu.make_async_copy(
                ones_v, degsp.at[rowc[i]], dsem[p]).wait()

        idx_fetch(0, 0)
        idx_fetch(1, 1)
        idx_wait(0)
        gather_launch(0, 0)

        def body(h, carry):
            for q in range(4):
                g = 4 * h + q
                p = q % 2
                i = q

                @pl.when((g + 1 < kpw) & (g >= 1))
                def _():
                    scatter_wait(1 - p, (i + 3) % 4)

                @pl.when(g + 1 < kpw)
                def _():
                    idx_wait((i + 1) % 4)
                    gather_launch(1 - p, (i + 1) % 4)

                gather_wait(p, i)
                scatter_launch(p, i)

                @pl.when(g + 2 < kpw)
                def _():
                    idx_fetch(g + 2, (i + 2) % 4)
            return carry

        lax.fori_loop(0, kpw // 4, body, 0)
        scatter_wait(0, 0)
        scatter_wait(1, 0)
        plsc.subcore_barrier()
        pltpu.sync_copy(
            agg.at[pl.ds(s * rows_per_tile, rows_per_tile)],
            agg_hbm.at[c, pl.ds(s * rows_per_tile, rows_per_tile)],
        )
        pltpu.sync_copy(
            degsp.at[pl.ds(s * rows_per_tile, rows_per_tile)],
            deg_hbm.at[c, pl.ds(s * rows_per_tile, rows_per_tile)],
        )

    return k(x, row1, col1, prow1, pcol1, z128, z16, ones)


def _tc_finish_body(x_ref, p_ref, d_ref, w_ref, b_ref, g_ref, be_ref, o_ref):
    s = x_ref[...] + p_ref[0] + p_ref[1]
    h = lax.dot_general(
        s, w_ref[...], (((1,), (1,)), ((), ())),
        preferred_element_type=jnp.float32,
    )
    deg = d_ref[0, :, 0:1] + d_ref[1, :, 0:1] + 1.0
    h = h + deg * b_ref[...]
    mean = jnp.mean(h, axis=1, keepdims=True)
    d = h - mean
    var = jnp.mean(d * d, axis=1, keepdims=True)
    y = d * lax.rsqrt(var + 1e-5) * g_ref[...] + be_ref[...]
    o_ref[...] = jnp.maximum(y, 0.0)


def _tc_finish(x, parts, degs, w, b2, gamma2, beta2):
    n = x.shape[0]
    dout = w.shape[0]
    bs = 2000
    grid = n // bs
    return pl.pallas_call(
        _tc_finish_body,
        grid=(grid,),
        in_specs=[
            pl.BlockSpec((bs, D), lambda i: (i, 0)),
            pl.BlockSpec((2, bs, D), lambda i: (0, i, 0)),
            pl.BlockSpec((2, bs, DW), lambda i: (0, i, 0)),
            pl.BlockSpec((dout, D), lambda i: (0, 0)),
            pl.BlockSpec((1, dout), lambda i: (0, 0)),
            pl.BlockSpec((1, dout), lambda i: (0, 0)),
            pl.BlockSpec((1, dout), lambda i: (0, 0)),
        ],
        out_specs=pl.BlockSpec((bs, dout), lambda i: (i, 0)),
        out_shape=jax.ShapeDtypeStruct((n, dout), jnp.float32),
    )(x, parts, degs, w, b2, gamma2, beta2)


def kernel(x, edge_index, W, b, gamma, beta):
    n, d_in = x.shape
    d_out = W.shape[0]
    e = edge_index.shape[1]
    ei = edge_index.astype(jnp.int32)

    info = plsc.get_sparse_core_info()
    ns = info.num_subcores
    tot = -(-e // CH)
    kpw = (-(-tot // (2 * ns)) + 3) // 4 * 4
    e_pad = 2 * ns * kpw * CH
    tot_real = e // CH
    nsynth = e_pad - e
    ar = jnp.arange(nsynth, dtype=jnp.int32)
    pad_col = jnp.concatenate([ei[1, tot_real * CH:], (ar * 131) % n])
    pad_row = jnp.concatenate(
        [ei[0, tot_real * CH:], n + ar % (NROWS_PAD - n)])

    z128 = jnp.zeros((CH, D), jnp.float32)
    z16 = jnp.zeros((NROWS_PAD // 16, DW), jnp.float32)
    ones = jnp.ones((CH, DW), jnp.float32)

    parts, degs = _sc_aggregate(
        x, ei[0], ei[1], pad_row, pad_col, z128, z16, ones, kpw, tot_real)
    return _tc_finish(
        x, parts, degs, W,
        b.reshape(1, d_out), gamma.reshape(1, d_out), beta.reshape(1, d_out))


# speedup vs baseline: 3.4470x; 1.0657x over previous
import functools

import jax
import jax.numpy as jnp
from jax import lax
from jax.experimental import pallas as pl
from jax.experimental.pallas import tpu as pltpu
from jax.experimental.pallas import tpu_sc as plsc

D = 128
CH = 128
DW = 16
NROWS_PAD = 10240


def _sc_aggregate(x, eif, prow1, pcol1, z128, z16, ones, kpw, tot_real, e):
    info = plsc.get_sparse_core_info()
    nc, ns = info.num_cores, info.num_subcores
    rows_per_tile = NROWS_PAD // ns
    mesh = plsc.VectorSubcoreMesh(core_axis_name="c", subcore_axis_name="s")

    @functools.partial(
        pl.kernel,
        mesh=mesh,
        compiler_params=pltpu.CompilerParams(use_tc_tiling_on_sc=False),
        out_type=[
            jax.ShapeDtypeStruct((nc, NROWS_PAD, D), jnp.float32),
            jax.ShapeDtypeStruct((nc, NROWS_PAD, DW), jnp.float32),
        ],
        scratch_types=(
            [pltpu.VMEM((CH,), jnp.int32)] * 4
            + [pltpu.VMEM((CH,), jnp.int32)] * 4
            + [pltpu.VMEM((CH, D), jnp.float32)] * 2
            + [pltpu.VMEM((CH, DW), jnp.float32)]
            + [pltpu.VMEM_SHARED((NROWS_PAD, D), jnp.float32)]
            + [pltpu.VMEM_SHARED((NROWS_PAD, DW), jnp.float32)]
            + [pltpu.SemaphoreType.DMA] * 10
        ),
    )
    def k(x_hbm, eif_hbm, prow_hbm, pcol_hbm, z128_hbm, z16_hbm,
          ones_hbm, agg_hbm, deg_hbm,
          cc0, cc1, cc2, cc3, rc0, rc1, rc2, rc3, rows0, rows1, ones_v,
          agg, degsp,
          is0, is1, is2, is3, gsem0, gsem1, ssem0, ssem1, dsem0, dsem1):
        c = lax.axis_index("c")
        s = lax.axis_index("s")
        wid = c * ns + s
        nw = nc * ns
        colc = (cc0, cc1, cc2, cc3)
        rowc = (rc0, rc1, rc2, rc3)
        rows = (rows0, rows1)
        isem = (is0, is1, is2, is3)
        gsem = (gsem0, gsem1)
        ssem = (ssem0, ssem1)
        dsem = (dsem0, dsem1)

        for t in range(rows_per_tile // CH):
            pltpu.sync_copy(
                z128_hbm, agg.at[pl.ds(s * rows_per_tile + t * CH, CH)])
        pltpu.sync_copy(z16_hbm, degsp.at[pl.ds(s * rows_per_tile, rows_per_tile)])
        pltpu.sync_copy(ones_hbm, ones_v)
        plsc.subcore_barrier()

        def idx_fetch(g, i):
            cid = wid + g * nw

            @pl.when(cid < tot_real)
            def _():
                e0 = cid * CH
                pltpu.async_copy(eif_hbm.at[pl.ds(e + e0, CH)], colc[i], isem[i])
                pltpu.async_copy(eif_hbm.at[pl.ds(e0, CH)], rowc[i], isem[i])

            @pl.when(cid >= tot_real)
            def _():
                e0 = (cid - tot_real) * CH
                pltpu.async_copy(pcol_hbm.at[pl.ds(e0, CH)], colc[i], isem[i])
                pltpu.async_copy(prow_hbm.at[pl.ds(e0, CH)], rowc[i], isem[i])

        def idx_wait(i):
            pltpu.make_async_copy(
                eif_hbm.at[pl.ds(0, CH)], colc[i], isem[i]).wait()
            pltpu.make_async_copy(
                eif_hbm.at[pl.ds(0, CH)], rowc[i], isem[i]).wait()

        def gather_launch(p, i):
            pltpu.async_copy(x_hbm.at[colc[i]], rows[p], gsem[p])

        def gather_wait(p, i):
            pltpu.make_async_copy(
                x_hbm.at[colc[i]], rows[p], gsem[p]).wait()

        def scatter_launch(p, i):
            pltpu.async_copy(rows[p], agg.at[rowc[i]], ssem[p], add=True)
            pltpu.async_copy(ones_v, degsp.at[rowc[i]], dsem[p], add=True)

        def scatter_wait(p, i):
            pltpu.make_async_copy(
                rows[p], agg.at[rowc[i]], ssem[p]).wait()
            pltpu.make_async_copy(
                ones_v, degsp.at[rowc[i]], dsem[p]).wait()

        idx_fetch(0, 0)
        idx_fetch(1, 1)
        idx_wait(0)
        gather_launch(0, 0)

        def body(h, carry):
            for q in range(4):
                g = 4 * h + q
                p = q % 2
                i = q

                @pl.when((g + 1 < kpw) & (g >= 1))
                def _():
                    scatter_wait(1 - p, (i + 3) % 4)

                @pl.when(g + 1 < kpw)
                def _():
                    idx_wait((i + 1) % 4)
                    gather_launch(1 - p, (i + 1) % 4)

                gather_wait(p, i)
                scatter_launch(p, i)

                @pl.when(g + 2 < kpw)
                def _():
                    idx_fetch(g + 2, (i + 2) % 4)
            return carry

        lax.fori_loop(0, kpw // 4, body, 0)
        scatter_wait(0, 0)
        scatter_wait(1, 0)
        plsc.subcore_barrier()
        pltpu.sync_copy(
            agg.at[pl.ds(s * rows_per_tile, rows_per_tile)],
            agg_hbm.at[c, pl.ds(s * rows_per_tile, rows_per_tile)],
        )
        pltpu.sync_copy(
            degsp.at[pl.ds(s * rows_per_tile, rows_per_tile)],
            deg_hbm.at[c, pl.ds(s * rows_per_tile, rows_per_tile)],
        )

    return k(x, eif, prow1, pcol1, z128, z16, ones)


def _tc_finish_body(x_ref, p_ref, d_ref, w_ref, b_ref, g_ref, be_ref, o_ref):
    s = x_ref[...] + p_ref[0] + p_ref[1]
    h = lax.dot_general(
        s, w_ref[...], (((1,), (1,)), ((), ())),
        preferred_element_type=jnp.float32,
    )
    deg = d_ref[0, :, 0:1] + d_ref[1, :, 0:1] + 1.0
    h = h + deg * b_ref[...]
    mean = jnp.mean(h, axis=1, keepdims=True)
    d = h - mean
    var = jnp.mean(d * d, axis=1, keepdims=True)
    y = d * lax.rsqrt(var + 1e-5) * g_ref[...] + be_ref[...]
    o_ref[...] = jnp.maximum(y, 0.0)


def _tc_finish(x, parts, degs, w, b2, gamma2, beta2):
    n = x.shape[0]
    dout = w.shape[0]
    bs = 2000
    grid = n // bs
    return pl.pallas_call(
        _tc_finish_body,
        grid=(grid,),
        in_specs=[
            pl.BlockSpec((bs, D), lambda i: (i, 0)),
            pl.BlockSpec((2, bs, D), lambda i: (0, i, 0)),
            pl.BlockSpec((2, bs, DW), lambda i: (0, i, 0)),
            pl.BlockSpec((dout, D), lambda i: (0, 0)),
            pl.BlockSpec((1, dout), lambda i: (0, 0)),
            pl.BlockSpec((1, dout), lambda i: (0, 0)),
            pl.BlockSpec((1, dout), lambda i: (0, 0)),
        ],
        out_specs=pl.BlockSpec((bs, dout), lambda i: (i, 0)),
        out_shape=jax.ShapeDtypeStruct((n, dout), jnp.float32),
    )(x, parts, degs, w, b2, gamma2, beta2)


def kernel(x, edge_index, W, b, gamma, beta):
    n, d_in = x.shape
    d_out = W.shape[0]
    e = edge_index.shape[1]
    ei = edge_index.astype(jnp.int32)

    info = plsc.get_sparse_core_info()
    ns = info.num_subcores
    tot = -(-e // CH)
    kpw = (-(-tot // (2 * ns)) + 3) // 4 * 4
    e_pad = 2 * ns * kpw * CH
    tot_real = e // CH
    nsynth = e_pad - e
    ar = jnp.arange(nsynth, dtype=jnp.int32)
    pad_col = jnp.concatenate([ei[1, tot_real * CH:], (ar * 131) % n])
    pad_row = jnp.concatenate(
        [ei[0, tot_real * CH:], n + ar % (NROWS_PAD - n)])

    z128 = jnp.zeros((CH, D), jnp.float32)
    z16 = jnp.zeros((NROWS_PAD // 16, DW), jnp.float32)
    ones = jnp.ones((CH, DW), jnp.float32)

    assert e % 8 == 0
    parts, degs = _sc_aggregate(
        x, ei.reshape(2 * e), pad_row, pad_col, z128, z16, ones,
        kpw, tot_real, e)
    return _tc_finish(
        x, parts, degs, W,
        b.reshape(1, d_out), gamma.reshape(1, d_out), beta.reshape(1, d_out))
